# Initial kernel scaffold; baseline (speedup 1.0000x reference)
import jax
import jax.numpy as jnp
from jax.experimental import pallas as pl


def kernel(neighbors, segment_ids, s, r, s_hist_dt, ent_embeds, rel_embeds):
    raise NotImplementedError("write your pallas kernel here")


# speedup vs baseline: 1.4015x; 1.4015x over previous
import jax
import jax.numpy as jnp
from jax import lax
from jax.experimental import pallas as pl
from jax.experimental.pallas import tpu as pltpu
from jax.experimental.pallas import tpu_sc as plsc

H = 256
B_SUBJ = 2048
SEQ = 20
TS = B_SUBJ * SEQ
TN = 200000

NC = 2
NS = 16
NW = NC * NS
L = 16

TILE_SEGS = TS // NW
SBT = 256
NSB = TILE_SEGS // SBT
K = 128
OUT_W = 3 * H

REP_ROWS = TS // NW
REP_CHUNKS = REP_ROWS // K


def _sc_body(nb_hbm, seg_hbm, offs_hbm, srep_hbm, rrep_hbm, ent_hbm,
             rel_hbm, out_hbm,
             acc_v, cnt_v, offs_v, nbr_v, seg_v, rows_v, sem):
    cid = lax.axis_index("c")
    sid = lax.axis_index("s")
    gid = sid * NC + cid

    iota = lax.iota(jnp.int32, L)
    onehot = jnp.where(iota == 0, 1.0, 0.0).astype(jnp.float32)
    zeros = jnp.zeros((L,), jnp.float32)

    pltpu.sync_copy(offs_hbm.at[gid], offs_v)
    offv = offs_v[pl.ds(0, 16)]
    offsc = [offv[j] for j in range(NSB + 1)]

    def _pick(idx):
        val = offsc[0]
        for j in range(1, NSB + 1):
            val = jnp.where(idx == j, offsc[j], val)
        return val

    def _subblock(u, _):
        base = gid * TILE_SEGS + u * SBT
        e0 = _pick(u)
        e1 = _pick(u + 1)

        def _zero(rr, _):
            for c in range(H // L):
                acc_v[rr, pl.ds(c * L, L)] = zeros
            return _
        lax.fori_loop(0, SBT + 1, _zero, None)

        def _zerocnt(rr, _):
            cnt_v[pl.ds(rr * L, L)] = zeros
            return _
        lax.fori_loop(0, (SBT + L) // L, _zerocnt, None)

        astart = (e0 // 8) * 8
        nchunks = jnp.maximum(0, (e1 - astart + K - 1) // K)

        def _chunk(t, _):
            a = pl.multiple_of(astart + t * K, 8)
            pltpu.sync_copy(seg_hbm.at[pl.ds(a, K)], seg_v)
            pltpu.sync_copy(nb_hbm.at[pl.ds(a, K)], nbr_v)
            pltpu.async_copy(ent_hbm.at[nbr_v], rows_v, sem).wait()

            def _group(g, _):
                sg = seg_v[pl.ds(g * L, L)]
                gv = (a + g * L) + iota
                valid = (gv >= e0) & (gv < e1)
                ls = jnp.clip(sg - base, 0, SBT)
                ls = jnp.where(valid, ls, SBT)
                for j in range(L):
                    lsj = ls[j]
                    er = g * L + j
                    for c in range(H // L):
                        plsc.addupdate(
                            acc_v.at[lsj, pl.ds(c * L, L)],
                            rows_v[er, pl.ds(c * L, L)])
                    plsc.addupdate(cnt_v.at[pl.ds(lsj, L)], onehot)
                return _

            lax.fori_loop(0, K // L, _group, None)
            return _

        lax.fori_loop(0, nchunks, _chunk, None)

        def _div(rb, _):
            cv = cnt_v[pl.ds(rb * L, L)]
            inv = 1.0 / jnp.maximum(cv, 1.0)
            for j in range(L):
                rr = rb * L + j
                invj = inv[j]
                for c in range(H // L):
                    acc_v[rr, pl.ds(c * L, L)] = (
                        acc_v[rr, pl.ds(c * L, L)] * invj)
            return _
        lax.fori_loop(0, SBT // L, _div, None)

        pltpu.sync_copy(acc_v.at[pl.ds(0, SBT)],
                        out_hbm.at[pl.ds(base, SBT), pl.ds(0, H)])
        return _

    lax.fori_loop(0, NSB, _subblock, None)

    def _rep(q, _):
        r0 = gid * REP_ROWS + q * K
        pltpu.sync_copy(srep_hbm.at[pl.ds(r0, K)], nbr_v)
        pltpu.async_copy(ent_hbm.at[nbr_v], rows_v, sem).wait()
        pltpu.sync_copy(rows_v, out_hbm.at[pl.ds(r0, K), pl.ds(H, H)])
        pltpu.sync_copy(rrep_hbm.at[pl.ds(r0, K)], nbr_v)
        pltpu.async_copy(rel_hbm.at[nbr_v], rows_v, sem).wait()
        pltpu.sync_copy(rows_v, out_hbm.at[pl.ds(r0, K), pl.ds(2 * H, H)])
        return _

    lax.fori_loop(0, REP_CHUNKS, _rep, None)


_sc_call = pl.kernel(
    _sc_body,
    out_type=jax.ShapeDtypeStruct((TS, OUT_W), jnp.float32),
    mesh=plsc.VectorSubcoreMesh(core_axis_name="c", subcore_axis_name="s"),
    scratch_types=[
        pltpu.VMEM((SBT + 1, H), jnp.float32),
        pltpu.VMEM((SBT + L,), jnp.float32),
        pltpu.VMEM((16,), jnp.int32),
        pltpu.VMEM((K,), jnp.int32),
        pltpu.VMEM((K,), jnp.int32),
        pltpu.VMEM((K, H), jnp.float32),
        pltpu.SemaphoreType.DMA,
    ],
)


def kernel(neighbors, segment_ids, s, r, s_hist_dt, ent_embeds, rel_embeds):
    neighbors = neighbors.astype(jnp.int32)
    segment_ids = segment_ids.astype(jnp.int32)
    nb_pad = jnp.concatenate([neighbors, jnp.zeros((K,), jnp.int32)])
    seg_pad = jnp.concatenate([segment_ids, jnp.zeros((K,), jnp.int32)])
    bounds = jnp.searchsorted(
        segment_ids, jnp.arange(0, TS + 1, SBT, dtype=jnp.int32)
    ).astype(jnp.int32)
    col = jnp.minimum(jnp.arange(16, dtype=jnp.int32), NSB)
    idx2d = jnp.arange(NW, dtype=jnp.int32)[:, None] * NSB + col[None, :]
    offs2d = bounds[idx2d]
    s_rep = jnp.repeat(s.astype(jnp.int32), SEQ)
    r_rep = jnp.repeat(r.astype(jnp.int32), SEQ)

    out2d = _sc_call(nb_pad, seg_pad, offs2d, s_rep, r_rep,
                     ent_embeds, rel_embeds)
    out3 = out2d.reshape(B_SUBJ, SEQ, OUT_W)
    return (out3, s_hist_dt, jnp.arange(B_SUBJ, dtype=jnp.int32), B_SUBJ)
